# baseline (speedup 1.0000x reference)
import functools

import jax
import jax.numpy as jnp
from jax import lax
from jax.experimental import pallas as pl
from jax.experimental.pallas import tpu as pltpu
from jax.experimental.pallas import tpu_sc as plsc

_LANES = 16
_NW = 32
_C = 32


@functools.lru_cache(maxsize=None)
def _build(N, S, D):
    assert N % _NW == 0
    b_per_w = N // _NW
    assert b_per_w % _C == 0
    n_chunks = b_per_w // _C
    mesh = plsc.VectorSubcoreMesh(core_axis_name="c", subcore_axis_name="s")

    @functools.partial(
        pl.kernel,
        out_type=jax.ShapeDtypeStruct((N, D), jnp.float32),
        mesh=mesh,
        scratch_types=[
            pltpu.VMEM((_C,), jnp.int32),
            pltpu.VMEM((_C, D), jnp.float32),
            pltpu.VMEM((_C, D), jnp.float32),
            pltpu.SemaphoreType.DMA,
        ],
    )
    def emb(ids_hbm, wte_hbm, wpe_hbm, out_hbm, idx_v, rows_v, pos_v, sem):
        wid = lax.axis_index("s") * 2 + lax.axis_index("c")
        w_base = wid * b_per_w

        def chunk_body(g, carry):
            base = w_base + g * _C
            s_base = lax.rem(base, S)
            pltpu.sync_copy(ids_hbm.at[pl.ds(base, _C)], idx_v)
            gather = pltpu.async_copy(wte_hbm.at[idx_v], rows_v, sem)
            pltpu.sync_copy(wpe_hbm.at[pl.ds(s_base, _C)], pos_v)
            gather.wait()

            def row_body(r, c2):
                def col_body(c, c3):
                    sl = pl.ds(c * _LANES, _LANES)
                    rows_v[r, sl] = rows_v[r, sl] + pos_v[r, sl]
                    return c3
                return lax.fori_loop(0, D // _LANES, col_body, c2, unroll=8)

            lax.fori_loop(0, _C, row_body, carry)
            pltpu.sync_copy(rows_v, out_hbm.at[pl.ds(base, _C)])
            return carry

        lax.fori_loop(0, n_chunks, chunk_body, 0)

    return emb


def kernel(input_ids, wte, wpe):
    B, S = input_ids.shape
    D = wte.shape[1]
    ids = input_ids.reshape(-1).astype(jnp.int32)
    out = _build(B * S, S, D)(ids, wte, wpe)
    return out.reshape(B, S, D)


# speedup vs baseline: 3.1730x; 3.1730x over previous
import functools

import jax
import jax.numpy as jnp
from jax import lax
from jax.experimental import pallas as pl
from jax.experimental.pallas import tpu as pltpu
from jax.experimental.pallas import tpu_sc as plsc

_LANES = 16
_NW = 32
_CS = 8
_NBUF = 3


@functools.lru_cache(maxsize=None)
def _build(B, S, D):
    assert S % _NW == 0
    s_per_w = S // _NW
    assert s_per_w % _CS == 0
    n_chunks = s_per_w // _CS
    C = B * _CS
    mesh = plsc.VectorSubcoreMesh(core_axis_name="c", subcore_axis_name="s")

    @functools.partial(
        pl.kernel,
        out_type=jax.ShapeDtypeStruct((B * S, D), jnp.float32),
        mesh=mesh,
        scratch_types=[
            pltpu.VMEM((B * s_per_w,), jnp.int32),
            pltpu.VMEM((_NBUF * C, D), jnp.float32),
            pltpu.VMEM((_NBUF * _CS, D), jnp.float32),
        ]
        + [pltpu.SemaphoreType.DMA] * (3 * _NBUF),
    )
    def emb(ids_hbm, wte_hbm, wpe_hbm, out_hbm, idx_all, rows_v, pos_v, *sems):
        gsem = sems[0:_NBUF]
        psem = sems[_NBUF:2 * _NBUF]
        wsem = sems[2 * _NBUF:3 * _NBUF]
        wid = lax.axis_index("s") * 2 + lax.axis_index("c")
        s0 = wid * s_per_w

        pltpu.sync_copy(ids_hbm.at[pl.ds(wid * (B * s_per_w), B * s_per_w)],
                        idx_all)

        def start(g, p):
            gd = pltpu.async_copy(
                wte_hbm.at[idx_all.at[pl.ds(g * C, C)]],
                rows_v.at[pl.ds(p * C, C)], gsem[p])
            pd = pltpu.async_copy(
                wpe_hbm.at[pl.ds(s0 + g * _CS, _CS)],
                pos_v.at[pl.ds(p * _CS, _CS)], psem[p])
            return gd, pd

        def write(g, p):
            return [
                pltpu.async_copy(
                    rows_v.at[pl.ds(p * C + b * _CS, _CS)],
                    out_hbm.at[pl.ds(b * S + s0 + g * _CS, _CS)], wsem[p])
                for b in range(B)
            ]

        gdesc = {}
        wdesc = {}
        for g in range(min(2, n_chunks)):
            gdesc[g] = start(g, g % _NBUF)

        for g in range(n_chunks):
            p = g % _NBUF
            gd, pd = gdesc.pop(g)
            gd.wait()
            pd.wait()
            if g + 2 < n_chunks:
                if g - 1 >= 0:
                    for d in wdesc.pop(g - 1):
                        d.wait()
                gdesc[g + 2] = start(g + 2, (g + 2) % _NBUF)

            @pl.loop(0, _CS * (D // _LANES), unroll=4)
            def _add(i):
                r = i >> 6
                sl = pl.ds((i & (D // _LANES - 1)) * _LANES, _LANES)
                v = pos_v[p * _CS + r, sl]
                for b in range(B):
                    plsc.addupdate(rows_v.at[p * C + b * _CS + r, sl], v)

            wdesc[g] = write(g, p)

        for g in sorted(wdesc):
            for d in wdesc.pop(g):
                d.wait()

    return emb


def kernel(input_ids, wte, wpe):
    B, S = input_ids.shape
    D = wte.shape[1]
    s_per_w = S // _NW
    ids = (input_ids.astype(jnp.int32)
           .reshape(B, _NW, s_per_w // _CS, _CS)
           .transpose(1, 2, 0, 3)
           .reshape(-1))
    out = _build(B, S, D)(ids, wte, wpe)
    return out.reshape(B, S, D)


# speedup vs baseline: 3.5235x; 1.1104x over previous
import functools

import jax
import jax.numpy as jnp
from jax import lax
from jax.experimental import pallas as pl
from jax.experimental.pallas import tpu as pltpu
from jax.experimental.pallas import tpu_sc as plsc

_LANES = 16
_NW = 32
_CS = 8
_NBUF = 3


@functools.lru_cache(maxsize=None)
def _build(B, S, D):
    assert S % _NW == 0
    s_per_w = S // _NW
    assert s_per_w % _CS == 0
    n_chunks = s_per_w // _CS
    C = B * _CS
    mesh = plsc.VectorSubcoreMesh(core_axis_name="c", subcore_axis_name="s")

    @functools.partial(
        pl.kernel,
        out_type=jax.ShapeDtypeStruct((B * S, D), jnp.float32),
        mesh=mesh,
        scratch_types=[
            pltpu.VMEM((B * s_per_w,), jnp.int32),
            pltpu.VMEM((_NBUF * C, D), jnp.float32),
            pltpu.VMEM((_NBUF * _CS, D), jnp.float32),
        ]
        + [pltpu.SemaphoreType.DMA] * (3 * _NBUF),
    )
    def emb(ids_hbm, wte_hbm, wpe_hbm, out_hbm, idx_all, rows_v, pos_v, *sems):
        gsem = sems[0:_NBUF]
        psem = sems[_NBUF:2 * _NBUF]
        wsem = sems[2 * _NBUF:3 * _NBUF]
        wid = lax.axis_index("s") * 2 + lax.axis_index("c")
        s0 = wid * s_per_w

        pltpu.sync_copy(ids_hbm.at[pl.ds(wid * (B * s_per_w), B * s_per_w)],
                        idx_all)

        def start(g, p):
            gd = pltpu.async_copy(
                wte_hbm.at[idx_all.at[pl.ds(g * C, C)]],
                rows_v.at[pl.ds(p * C, C)], gsem[p])
            pd = pltpu.async_copy(
                wpe_hbm.at[pl.ds(s0 + g * _CS, _CS)],
                pos_v.at[pl.ds(p * _CS, _CS)], psem[p])
            return gd, pd

        def write(g, p):
            return [
                pltpu.async_copy(
                    rows_v.at[pl.ds(p * C + b * _CS, _CS)],
                    out_hbm.at[pl.ds(b * S + s0 + g * _CS, _CS)], wsem[p])
                for b in range(B)
            ]

        gdesc = {}
        wdesc = {}
        for g in range(min(2, n_chunks)):
            gdesc[g] = start(g, g % _NBUF)

        for g in range(n_chunks):
            p = g % _NBUF
            gd, pd = gdesc.pop(g)
            gd.wait()
            pd.wait()

            @pl.loop(0, _CS * (D // _LANES), unroll=8)
            def _add(i):
                r = i // (D // _LANES)
                sl = pl.ds((i % (D // _LANES)) * _LANES, _LANES)
                v = pos_v[p * _CS + r, sl]
                for b in range(B):
                    plsc.addupdate(rows_v.at[p * C + b * _CS + r, sl], v)

            wdesc[g] = write(g, p)
            if g + 2 < n_chunks:
                if g - 1 >= 0:
                    for d in wdesc.pop(g - 1):
                        d.wait()
                gdesc[g + 2] = start(g + 2, (g + 2) % _NBUF)

        for g in sorted(wdesc):
            for d in wdesc.pop(g):
                d.wait()

    return emb


def kernel(input_ids, wte, wpe):
    B, S = input_ids.shape
    D = wte.shape[1]
    s_per_w = S // _NW
    ids = (input_ids.astype(jnp.int32)
           .reshape(B, _NW, s_per_w // _CS, _CS)
           .transpose(1, 2, 0, 3)
           .reshape(-1))
    out = _build(B, S, D)(ids, wte, wpe)
    return out.reshape(B, S, D)


# speedup vs baseline: 3.5302x; 1.0019x over previous
import functools

import jax
import jax.numpy as jnp
from jax import lax
from jax.experimental import pallas as pl
from jax.experimental.pallas import tpu as pltpu
from jax.experimental.pallas import tpu_sc as plsc

_LANES = 16
_NW = 32
_CS = 8
_NBUF = 3


@functools.lru_cache(maxsize=None)
def _build(B, S, D):
    assert S % _NW == 0
    s_per_w = S // _NW
    assert s_per_w % _CS == 0
    n_chunks = s_per_w // _CS
    C = B * _CS
    mesh = plsc.VectorSubcoreMesh(core_axis_name="c", subcore_axis_name="s")

    @functools.partial(
        pl.kernel,
        out_type=jax.ShapeDtypeStruct((B * S, D), jnp.float32),
        mesh=mesh,
        scratch_types=[
            pltpu.VMEM((B * s_per_w,), jnp.int32),
            pltpu.VMEM((_NBUF * C, D), jnp.float32),
            pltpu.VMEM((_NBUF * _CS, D), jnp.float32),
        ]
        + [pltpu.SemaphoreType.DMA] * (3 * _NBUF),
    )
    def emb(ids_hbm, wte_hbm, wpe_hbm, out_hbm, idx_all, rows_v, pos_v, *sems):
        gsem = sems[0:_NBUF]
        psem = sems[_NBUF:2 * _NBUF]
        wsem = sems[2 * _NBUF:3 * _NBUF]
        wid = lax.axis_index("s") * 2 + lax.axis_index("c")
        s0 = wid * s_per_w

        pltpu.sync_copy(ids_hbm.at[pl.ds(wid * (B * s_per_w), B * s_per_w)],
                        idx_all)

        def start(g, p):
            h = C // 2
            gd = pltpu.async_copy(
                wte_hbm.at[idx_all.at[pl.ds(g * C, h)]],
                rows_v.at[pl.ds(p * C, h)], gsem[p])
            gd2 = pltpu.async_copy(
                wte_hbm.at[idx_all.at[pl.ds(g * C + h, h)]],
                rows_v.at[pl.ds(p * C + h, h)], gsem[p])
            pd = pltpu.async_copy(
                wpe_hbm.at[pl.ds(s0 + g * _CS, _CS)],
                pos_v.at[pl.ds(p * _CS, _CS)], psem[p])
            return gd, gd2, pd

        def write(g, p):
            return [
                pltpu.async_copy(
                    rows_v.at[pl.ds(p * C + b * _CS, _CS)],
                    out_hbm.at[pl.ds(b * S + s0 + g * _CS, _CS)], wsem[p])
                for b in range(B)
            ]

        gdesc = {}
        wdesc = {}
        for g in range(min(2, n_chunks)):
            gdesc[g] = start(g, g % _NBUF)

        for g in range(n_chunks):
            p = g % _NBUF
            gd, gd2, pd = gdesc.pop(g)
            gd.wait()
            gd2.wait()
            pd.wait()

            @pl.loop(0, _CS * (D // _LANES), unroll=8)
            def _add(i):
                r = i // (D // _LANES)
                sl = pl.ds((i % (D // _LANES)) * _LANES, _LANES)
                v = pos_v[p * _CS + r, sl]
                for b in range(B):
                    plsc.addupdate(rows_v.at[p * C + b * _CS + r, sl], v)

            wdesc[g] = write(g, p)
            if g + 2 < n_chunks:
                if g - 1 >= 0:
                    for d in wdesc.pop(g - 1):
                        d.wait()
                gdesc[g + 2] = start(g + 2, (g + 2) % _NBUF)

        for g in sorted(wdesc):
            for d in wdesc.pop(g):
                d.wait()

    return emb


def kernel(input_ids, wte, wpe):
    B, S = input_ids.shape
    D = wte.shape[1]
    s_per_w = S // _NW
    ids = (input_ids.astype(jnp.int32)
           .reshape(B, _NW, s_per_w // _CS, _CS)
           .transpose(1, 2, 0, 3)
           .reshape(-1))
    out = _build(B, S, D)(ids, wte, wpe)
    return out.reshape(B, S, D)
